# Initial kernel scaffold; baseline (speedup 1.0000x reference)
import jax
import jax.numpy as jnp
from jax.experimental import pallas as pl


def kernel(local, pair, pos, extra_pos, extra_pair, extra_pair_mask, neighbours, mask, W_e1, Wg1, Wv1, Wo1, ln1_scale, ln1_offset, W_e2, Wg2, Wv2, Wo2, ln3_scale, ln3_offset, Wg3, Wv3, Wo3, ln4_scale, ln4_offset):
    raise NotImplementedError("write your pallas kernel here")


# speedup vs baseline: 2.5322x; 2.5322x over previous
import functools

import jax
import jax.numpy as jnp
from jax import lax
from jax.experimental import pallas as pl
from jax.experimental.pallas import tpu as pltpu
from jax.experimental.pallas import tpu_sc as plsc

N = 10000
K = 16
KE = 8
D = 128
TW = 128

NC, NS = 2, 16
NW = NC * NS
TOTAL = N * K
PER_W = TOTAL // NW
CHUNK = 40
NCHUNK = PER_W // CHUNK
GROUP = 5
NGROUP = NCHUNK // GROUP
PER_WP = PER_W + 8
NVEC = PER_WP // 16
NPADM = N + 8

B = 200
GRID = N // B
BK = B * K
BE = B * KE


def _sc_gather(table, idx, maskt):
    mesh = plsc.VectorSubcoreMesh(core_axis_name="c", subcore_axis_name="s")

    @functools.partial(
        pl.kernel,
        mesh=mesh,
        out_type=[
            jax.ShapeDtypeStruct((TOTAL, TW), jnp.float32),
            jax.ShapeDtypeStruct((TOTAL,), jnp.float32),
        ],
        scratch_types=[
            pltpu.VMEM((NCHUNK, CHUNK), jnp.int32),
            pltpu.VMEM((PER_W,), jnp.float32),
        ] + [pltpu.VMEM((CHUNK, TW), jnp.float32) for _ in range(GROUP)] + [
            pltpu.SemaphoreType.DMA,
            pltpu.SemaphoreType.DMA,
            pltpu.SemaphoreType.DMA,
        ],
    )
    def gk(table_hbm, idx_hbm, maskt_hbm, out_hbm, mout_hbm,
           idx_v, mout_v, *rest):
        bufs = rest[:GROUP]
        sem_g, sem_w, sem_m = rest[GROUP], rest[GROUP + 1], rest[GROUP + 2]
        wid = lax.axis_index("s") * NC + lax.axis_index("c")
        pltpu.sync_copy(idx_hbm.at[wid], idx_v)
        base = wid * PER_W

        def body(g, carry):
            j0 = g * GROUP
            gathers = [
                pltpu.async_copy(table_hbm.at[idx_v.at[j0 + b]], bufs[b], sem_g)
                for b in range(GROUP)
            ]
            mgathers = [
                pltpu.async_copy(
                    maskt_hbm.at[idx_v.at[j0 + b]],
                    mout_v.at[pl.ds((j0 + b) * CHUNK, CHUNK)], sem_m)
                for b in range(GROUP)
            ]
            writes = []
            for b in range(GROUP):
                gathers[b].wait()
                writes.append(pltpu.async_copy(
                    bufs[b], out_hbm.at[pl.ds(base + (j0 + b) * CHUNK, CHUNK)],
                    sem_w))
            for m in mgathers:
                m.wait()
            for w in writes:
                w.wait()
            return carry

        lax.fori_loop(0, NGROUP, body, 0)
        pltpu.sync_copy(mout_v, mout_hbm.at[pl.ds(base, PER_W)])

    return gk(table, idx, maskt)


def _swish(x):
    return x * (1.0 / (1.0 + jnp.exp(-x)))


def _mm(a, b):
    return jax.lax.dot_general(
        a, b, (((1,), (0,)), ((), ())),
        precision=jax.lax.Precision.DEFAULT,
        preferred_element_type=jnp.float32,
    )


def _ln(x, scale, offset):
    mu = jnp.mean(x, axis=-1, keepdims=True)
    d = x - mu
    var = jnp.mean(d * d, axis=-1, keepdims=True)
    return d * jax.lax.rsqrt(var + 1e-5) * scale + offset


def _tc_body(local_ref, pair_ref, extra_ref, gath_ref, pmask_ref, emask_ref,
             w1p, w1c, w1n, w1e, wo1,
             w2p, w2c, w2n, w2e, wo2,
             w3, wo3, lnp,
             out_local, out_pair, out_extra):
    lb = local_ref[...]
    pr = pair_ref[...].reshape(BK, D)
    ex = extra_ref[...].reshape(BE, D)
    g = gath_ref[...].reshape(BK, D)
    pm = pmask_ref[...]
    em = emask_ref[...]
    w1p, w1c, w1n, w1e, wo1 = w1p[...], w1c[...], w1n[...], w1e[...], wo1[...]
    w2p, w2c, w2n, w2e, wo2 = w2p[...], w2c[...], w2n[...], w2e[...], wo2[...]
    w3, wo3, lnp = w3[...], wo3[...], lnp[...]

    ln1s, ln1o = lnp[0:1, :], lnp[1:2, :]
    ln3s, ln3o = lnp[2:3, :], lnp[3:4, :]
    ln4s, ln4o = lnp[4:5, :], lnp[5:6, :]

    cc1 = _mm(lb, w1c)
    x1 = (_mm(pr, w1p) + _mm(g, w1n)).reshape(B, K, 2 * D) \
        + lax.broadcast_in_dim(cc1, (B, K, 2 * D), (0, 2))
    h1 = (_swish(x1[:, :, :D]) * x1[:, :, D:]).reshape(BK, D)
    pair_new = _ln(pr + _mm(h1, wo1), ln1s, ln1o)

    x1e = _mm(ex, w1e).reshape(B, KE, 2 * D) \
        + lax.broadcast_in_dim(cc1, (B, KE, 2 * D), (0, 2))
    h1e = (_swish(x1e[:, :, :D]) * x1e[:, :, D:]).reshape(BE, D)
    extra_new = _ln(ex + _mm(h1e, wo1), ln1s, ln1o)

    cc2 = _mm(lb, w2c)
    x2 = (_mm(pair_new, w2p) + _mm(g, w2n)).reshape(B, K, 2 * D) \
        + lax.broadcast_in_dim(cc2, (B, K, 2 * D), (0, 2))
    h2 = _swish(x2[:, :, :D]) * x2[:, :, D:]
    hsum = jnp.sum(h2 * pm, axis=1)

    x2e = _mm(extra_new, w2e).reshape(B, KE, 2 * D) \
        + lax.broadcast_in_dim(cc2, (B, KE, 2 * D), (0, 2))
    h2e = _swish(x2e[:, :, :D]) * x2e[:, :, D:]
    hsum = hsum + jnp.sum(h2e * em, axis=1)

    msg = _mm(hsum, wo2) * (1.0 / (K + KE))
    loc2 = _ln(lb + msg, ln3s, ln3o)

    x3 = _mm(loc2, w3)
    h3 = _swish(x3[:, :D]) * x3[:, D:]
    loc3 = _ln(loc2 + _mm(h3, wo3), ln4s, ln4o)

    out_local[...] = loc3
    out_pair[...] = pair_new.reshape(B, K, D)
    out_extra[...] = extra_new.reshape(B, KE, D)


def kernel(local, pair, pos, extra_pos, extra_pair, extra_pair_mask, neighbours, mask,
           W_e1, Wg1, Wv1, Wo1, ln1_scale, ln1_offset,
           W_e2, Wg2, Wv2, Wo2, ln3_scale, ln3_offset,
           Wg3, Wv3, Wo3, ln4_scale, ln4_offset):
    f32 = jnp.float32

    table = jnp.concatenate([local, local[-1:]], axis=0)
    idx = jnp.where(neighbours < 0, N, neighbours).astype(jnp.int32)
    idx3 = idx.reshape(NW, NCHUNK, CHUNK)
    maskt = jnp.concatenate([mask, jnp.zeros((NPADM - N,), f32)])

    gathered, maskvals = _sc_gather(table, idx3, maskt)
    gathered = gathered.reshape(N, K, TW)
    pmask3 = maskvals.reshape(N, K, 1)

    def split3(w):
        return w[:D], w[D:2 * D], w[2 * D:]

    A1g, B1g, C1g = split3(Wg1)
    A1v, B1v, C1v = split3(Wv1)
    A2g, B2g, C2g = split3(Wg2)
    A2v, B2v, C2v = split3(Wv2)
    w1p = jnp.concatenate([A1g, A1v], axis=1)
    w1c = jnp.concatenate([B1g, B1v], axis=1)
    w1n = jnp.concatenate([C1g, C1v], axis=1)
    w1e = jnp.concatenate([A1g + W_e1 @ C1g, A1v + W_e1 @ C1v], axis=1)
    w2p = jnp.concatenate([A2g, A2v], axis=1)
    w2c = jnp.concatenate([B2g, B2v], axis=1)
    w2n = jnp.concatenate([C2g, C2v], axis=1)
    w2e = jnp.concatenate([A2g + W_e2 @ C2g, A2v + W_e2 @ C2v], axis=1)
    w3 = jnp.concatenate([Wg3, Wv3], axis=1)
    lnp = jnp.stack([ln1_scale, ln1_offset, ln3_scale, ln3_offset,
                     ln4_scale, ln4_offset])

    emask3 = extra_pair_mask.astype(f32)[:, :, None]

    def blk(shape):
        return pl.BlockSpec(shape, lambda i: (i,) + (0,) * (len(shape) - 1))

    def full(shape):
        return pl.BlockSpec(shape, lambda i: (0,) * len(shape))

    out_local, out_pair, out_extra = pl.pallas_call(
        _tc_body,
        grid=(GRID,),
        in_specs=[
            blk((B, D)),
            blk((B, K, D)),
            blk((B, KE, D)),
            blk((B, K, TW)),
            blk((B, K, 1)),
            blk((B, KE, 1)),
            full((D, 2 * D)),
            full((D, 2 * D)),
            full((D, 2 * D)),
            full((D, 2 * D)),
            full((D, D)),
            full((D, 2 * D)),
            full((D, 2 * D)),
            full((D, 2 * D)),
            full((D, 2 * D)),
            full((D, D)),
            full((D, 2 * D)),
            full((D, D)),
            full((6, D)),
        ],
        out_specs=[
            blk((B, D)),
            blk((B, K, D)),
            blk((B, KE, D)),
        ],
        out_shape=[
            jax.ShapeDtypeStruct((N, D), f32),
            jax.ShapeDtypeStruct((N, K, D), f32),
            jax.ShapeDtypeStruct((N, KE, D), f32),
        ],
        compiler_params=pltpu.CompilerParams(
            dimension_semantics=("arbitrary",),
        ),
    )(local, pair, extra_pair, gathered, pmask3, emask3,
      w1p, w1c, w1n, w1e, Wo1, w2p, w2c, w2n, w2e, Wo2, w3, Wo3, lnp)

    return (out_local, out_pair, out_extra)
